# baseline (speedup 1.0000x reference)
import functools
import jax
import jax.numpy as jnp
from jax import lax
from jax.experimental import pallas as pl
from jax.experimental.pallas import tpu as pltpu
from jax.experimental.pallas import tpu_sc as plsc

N = 10000
NPAD = 10240
F = 128
C = 64
NE = 320000
CHUNK = 128
NCHUNK = 80
HALF = NCHUNK // 2
EPT = NCHUNK * CHUNK
NE_PAD = EPT * 32
ROWS_PT = NPAD // 16
BM = 256


def _make_prop(D):
    mesh = plsc.VectorSubcoreMesh(core_axis_name="c", subcore_axis_name="s")

    @functools.partial(
        pl.kernel,
        out_type=jax.ShapeDtypeStruct((2, NPAD, D), jnp.float32),
        mesh=mesh,
        compiler_params=pltpu.CompilerParams(use_tc_tiling_on_sc=False),
        scratch_types=[
            pltpu.VMEM((HALF, CHUNK), jnp.int32),
            pltpu.VMEM((HALF, CHUNK), jnp.int32),
            pltpu.VMEM((CHUNK, D), jnp.float32),
            pltpu.VMEM((CHUNK, D), jnp.float32),
            pltpu.VMEM_SHARED((NPAD, D), jnp.float32),
            pltpu.SemaphoreType.DMA,
            pltpu.SemaphoreType.DMA,
        ],
    )
    def prop(h_hbm, src_hbm, dst_hbm, z_hbm, out_hbm, src_v, dst_v, rows_a,
             rows_b, agg_sh, sem_a, sem_b):
        c = lax.axis_index("c")
        s = lax.axis_index("s")
        wid = c * 16 + s
        pltpu.sync_copy(z_hbm, rows_a)
        for j in range(ROWS_PT // CHUNK):
            pltpu.sync_copy(
                rows_a, agg_sh.at[pl.ds(s * ROWS_PT + j * CHUNK, CHUNK)])
        plsc.subcore_barrier()

        for hf in range(2):
            pltpu.sync_copy(src_hbm.at[wid].at[hf], src_v)
            pltpu.sync_copy(dst_hbm.at[wid].at[hf], dst_v)
            pltpu.async_copy(h_hbm.at[src_v.at[0]], rows_a, sem_a)

            def body(k, carry):
                j = 2 * k
                pltpu.async_copy(h_hbm.at[src_v.at[j + 1]], rows_b, sem_b)
                pltpu.make_async_copy(
                    h_hbm.at[src_v.at[j]], rows_a, sem_a).wait()
                pltpu.sync_copy(rows_a, agg_sh.at[dst_v.at[j]], add=True)
                pltpu.async_copy(h_hbm.at[src_v.at[j + 2]], rows_a, sem_a)
                pltpu.make_async_copy(
                    h_hbm.at[src_v.at[j + 1]], rows_b, sem_b).wait()
                pltpu.sync_copy(rows_b, agg_sh.at[dst_v.at[j + 1]], add=True)
                return carry

            lax.fori_loop(0, (HALF - 2) // 2, body, 0)
            pltpu.async_copy(h_hbm.at[src_v.at[HALF - 1]], rows_b, sem_b)
            pltpu.make_async_copy(
                h_hbm.at[src_v.at[HALF - 2]], rows_a, sem_a).wait()
            pltpu.sync_copy(rows_a, agg_sh.at[dst_v.at[HALF - 2]], add=True)
            pltpu.make_async_copy(
                h_hbm.at[src_v.at[HALF - 1]], rows_b, sem_b).wait()
            pltpu.sync_copy(rows_b, agg_sh.at[dst_v.at[HALF - 1]], add=True)
        plsc.subcore_barrier()
        for j in range(ROWS_PT // CHUNK):
            off = s * ROWS_PT + j * CHUNK
            pltpu.sync_copy(agg_sh.at[pl.ds(off, CHUNK)], rows_a)
            pltpu.sync_copy(rows_a, out_hbm.at[c].at[pl.ds(off, CHUNK)])

    return prop


def _make_deg():
    D = 1
    mesh = plsc.VectorSubcoreMesh(core_axis_name="c", subcore_axis_name="s")

    @functools.partial(
        pl.kernel,
        out_type=jax.ShapeDtypeStruct((2, NPAD, D), jnp.float32),
        mesh=mesh,
        compiler_params=pltpu.CompilerParams(use_tc_tiling_on_sc=False),
        scratch_types=[
            pltpu.VMEM((HALF, CHUNK), jnp.int32),
            pltpu.VMEM((CHUNK, D), jnp.float32),
            pltpu.VMEM((CHUNK, D), jnp.float32),
            pltpu.VMEM_SHARED((NPAD, D), jnp.float32),
        ],
    )
    def deg(dst_hbm, z_hbm, ones_hbm, out_hbm, dst_v, zrows_v, ones_v, deg_sh):
        c = lax.axis_index("c")
        s = lax.axis_index("s")
        wid = c * 16 + s
        pltpu.sync_copy(z_hbm, zrows_v)
        pltpu.sync_copy(ones_hbm, ones_v)
        for j in range(ROWS_PT // CHUNK):
            pltpu.sync_copy(
                zrows_v, deg_sh.at[pl.ds(s * ROWS_PT + j * CHUNK, CHUNK)])
        plsc.subcore_barrier()

        def body(j, carry):
            pltpu.sync_copy(ones_v, deg_sh.at[dst_v.at[j]], add=True)
            return carry

        for hf in range(2):
            pltpu.sync_copy(dst_hbm.at[wid].at[hf], dst_v)
            lax.fori_loop(0, HALF, body, 0)
        plsc.subcore_barrier()
        for j in range(ROWS_PT // CHUNK):
            off = s * ROWS_PT + j * CHUNK
            pltpu.sync_copy(deg_sh.at[pl.ds(off, CHUNK)], zrows_v)
            pltpu.sync_copy(zrows_v, out_hbm.at[c].at[pl.ds(off, CHUNK)])

    return deg


_prop128 = _make_prop(F)
_prop64 = _make_prop(C)
_deg = _make_deg()


def _dis_of(degb):
    return lax.rsqrt(degb[0, :, :1] + degb[1, :, :1] + 1.0)


def _tc1_body(x_ref, deg_ref, w1_ref, o_ref):
    dis = _dis_of(deg_ref[...])
    h = jnp.dot(x_ref[...], w1_ref[...], preferred_element_type=jnp.float32)
    o_ref[...] = h * dis


def _tc2_body(agg_ref, hs_ref, deg_ref, b1_ref, w2_ref, o_ref):
    dis = _dis_of(deg_ref[...])
    a = agg_ref[0] + agg_ref[1] + hs_ref[...]
    o = a * dis + b1_ref[...]
    o = jnp.maximum(o, 0.0)
    h2 = jnp.dot(o, w2_ref[...], preferred_element_type=jnp.float32)
    o_ref[...] = h2 * dis


def _tc3_body(agg_ref, hs_ref, deg_ref, b2_ref, o_ref):
    dis = _dis_of(deg_ref[...])
    a = agg_ref[0] + agg_ref[1] + hs_ref[...]
    o = a * dis + b2_ref[...]
    m = jnp.max(o, axis=1, keepdims=True)
    z = o - m
    lse = jnp.log(jnp.sum(jnp.exp(z), axis=1, keepdims=True))
    o_ref[...] = z - lse


_GRID = NPAD // BM


def _row_spec(d):
    return pl.BlockSpec((BM, d), lambda i: (i, 0))


def _p2_spec(d):
    return pl.BlockSpec((2, BM, d), lambda i: (0, i, 0))


def _full_spec(r, c_):
    return pl.BlockSpec((r, c_), lambda i: (0, 0))


_tc1 = pl.pallas_call(
    _tc1_body,
    grid=(_GRID,),
    in_specs=[_row_spec(F), _p2_spec(1), _full_spec(F, F)],
    out_specs=_row_spec(F),
    out_shape=jax.ShapeDtypeStruct((NPAD, F), jnp.float32),
)

_tc2 = pl.pallas_call(
    _tc2_body,
    grid=(_GRID,),
    in_specs=[_p2_spec(F), _row_spec(F), _p2_spec(1), _full_spec(1, F),
              _full_spec(F, C)],
    out_specs=_row_spec(C),
    out_shape=jax.ShapeDtypeStruct((NPAD, C), jnp.float32),
)

_tc3 = pl.pallas_call(
    _tc3_body,
    grid=(_GRID,),
    in_specs=[_p2_spec(C), _row_spec(C), _p2_spec(1), _full_spec(1, C)],
    out_specs=_row_spec(C),
    out_shape=jax.ShapeDtypeStruct((NPAD, C), jnp.float32),
)


@jax.jit
def kernel(x, edge_index, W1, b1, W2, b2):
    src = edge_index[0].astype(jnp.int32)
    dst = edge_index[1].astype(jnp.int32)
    pad = NE_PAD - NE
    pad_dst = N + jnp.arange(pad, dtype=jnp.int32) % (NPAD - N)
    src3 = jnp.concatenate(
        [src, jnp.zeros((pad,), jnp.int32)]).reshape(32, 2, HALF, CHUNK)
    dst3 = jnp.concatenate([dst, pad_dst]).reshape(32, 2, HALF, CHUNK)
    xp = jnp.pad(x, ((0, NPAD - N), (0, 0)))
    z128 = jnp.zeros((CHUNK, F), jnp.float32)
    z64 = jnp.zeros((CHUNK, C), jnp.float32)
    z1 = jnp.zeros((CHUNK, 1), jnp.float32)
    ones1 = jnp.ones((CHUNK, 1), jnp.float32)

    degp = _deg(dst3, z1, ones1)
    h1s = _tc1(xp, degp, W1)
    agg1 = _prop128(h1s, src3, dst3, z128)
    h2s = _tc2(agg1, h1s, degp, b1.reshape(1, F), W2)
    agg2 = _prop64(h2s, src3, dst3, z64)
    out = _tc3(agg2, h2s, degp, b2.reshape(1, C))
    return out[:N]


# speedup vs baseline: 1.1032x; 1.1032x over previous
import functools
import jax
import jax.numpy as jnp
from jax import lax
from jax.experimental import pallas as pl
from jax.experimental.pallas import tpu as pltpu
from jax.experimental.pallas import tpu_sc as plsc

N = 10000
NPAD = 10240
F = 128
C = 64
NE = 320000
CHUNK = 128
NCHUNK = 80
HALF = NCHUNK // 2
EPT = NCHUNK * CHUNK
NE_PAD = EPT * 32
ROWS_PT = NPAD // 16
BM = 256


def _make_prop(D):
    mesh = plsc.VectorSubcoreMesh(core_axis_name="c", subcore_axis_name="s")

    @functools.partial(
        pl.kernel,
        out_type=jax.ShapeDtypeStruct((2, NPAD, D), jnp.float32),
        mesh=mesh,
        compiler_params=pltpu.CompilerParams(use_tc_tiling_on_sc=False),
        scratch_types=[
            pltpu.VMEM((HALF, CHUNK), jnp.int32),
            pltpu.VMEM((HALF, CHUNK), jnp.int32),
            pltpu.VMEM((CHUNK, D), jnp.float32),
            pltpu.VMEM((CHUNK, D), jnp.float32),
            pltpu.VMEM_SHARED((NPAD, D), jnp.float32),
            pltpu.SemaphoreType.DMA,
            pltpu.SemaphoreType.DMA,
        ],
    )
    def prop(h_hbm, src_hbm, dst_hbm, z_hbm, out_hbm, src_v, dst_v, rows_a,
             rows_b, agg_sh, sem_a, sem_b):
        c = lax.axis_index("c")
        s = lax.axis_index("s")
        wid = c * 16 + s
        pltpu.sync_copy(z_hbm, rows_a)
        for j in range(ROWS_PT // CHUNK):
            pltpu.sync_copy(
                rows_a, agg_sh.at[pl.ds(s * ROWS_PT + j * CHUNK, CHUNK)])
        plsc.subcore_barrier()

        for hf in range(2):
            pltpu.sync_copy(src_hbm.at[wid].at[hf], src_v)
            pltpu.sync_copy(dst_hbm.at[wid].at[hf], dst_v)
            pltpu.async_copy(h_hbm.at[src_v.at[0]], rows_a, sem_a)

            def body(k, carry):
                j = 2 * k
                pltpu.async_copy(h_hbm.at[src_v.at[j + 1]], rows_b, sem_b)
                pltpu.make_async_copy(
                    h_hbm.at[src_v.at[j]], rows_a, sem_a).wait()
                pltpu.sync_copy(rows_a, agg_sh.at[dst_v.at[j]], add=True)
                pltpu.async_copy(h_hbm.at[src_v.at[j + 2]], rows_a, sem_a)
                pltpu.make_async_copy(
                    h_hbm.at[src_v.at[j + 1]], rows_b, sem_b).wait()
                pltpu.sync_copy(rows_b, agg_sh.at[dst_v.at[j + 1]], add=True)
                return carry

            lax.fori_loop(0, (HALF - 2) // 2, body, 0)
            pltpu.async_copy(h_hbm.at[src_v.at[HALF - 1]], rows_b, sem_b)
            pltpu.make_async_copy(
                h_hbm.at[src_v.at[HALF - 2]], rows_a, sem_a).wait()
            pltpu.sync_copy(rows_a, agg_sh.at[dst_v.at[HALF - 2]], add=True)
            pltpu.make_async_copy(
                h_hbm.at[src_v.at[HALF - 1]], rows_b, sem_b).wait()
            pltpu.sync_copy(rows_b, agg_sh.at[dst_v.at[HALF - 1]], add=True)
        plsc.subcore_barrier()
        for j in range(ROWS_PT // CHUNK):
            off = s * ROWS_PT + j * CHUNK
            pltpu.sync_copy(agg_sh.at[pl.ds(off, CHUNK)], rows_a)
            pltpu.sync_copy(rows_a, out_hbm.at[c].at[pl.ds(off, CHUNK)])

    return prop


def _make_deg():
    D = 8
    mesh = plsc.VectorSubcoreMesh(core_axis_name="c", subcore_axis_name="s")

    @functools.partial(
        pl.kernel,
        out_type=jax.ShapeDtypeStruct((2, NPAD, D), jnp.float32),
        mesh=mesh,
        compiler_params=pltpu.CompilerParams(use_tc_tiling_on_sc=False),
        scratch_types=[
            pltpu.VMEM((HALF, CHUNK), jnp.int32),
            pltpu.VMEM((CHUNK, D), jnp.float32),
            pltpu.VMEM((CHUNK, D), jnp.float32),
            pltpu.VMEM_SHARED((NPAD, D), jnp.float32),
        ],
    )
    def deg(dst_hbm, z_hbm, ones_hbm, out_hbm, dst_v, zrows_v, ones_v, deg_sh):
        c = lax.axis_index("c")
        s = lax.axis_index("s")
        wid = c * 16 + s
        pltpu.sync_copy(z_hbm, zrows_v)
        pltpu.sync_copy(ones_hbm, ones_v)
        for j in range(ROWS_PT // CHUNK):
            pltpu.sync_copy(
                zrows_v, deg_sh.at[pl.ds(s * ROWS_PT + j * CHUNK, CHUNK)])
        plsc.subcore_barrier()

        def body(j, carry):
            pltpu.sync_copy(ones_v, deg_sh.at[dst_v.at[j]], add=True)
            return carry

        for hf in range(2):
            pltpu.sync_copy(dst_hbm.at[wid].at[hf], dst_v)
            lax.fori_loop(0, HALF, body, 0)
        plsc.subcore_barrier()
        for j in range(ROWS_PT // CHUNK):
            off = s * ROWS_PT + j * CHUNK
            pltpu.sync_copy(deg_sh.at[pl.ds(off, CHUNK)], zrows_v)
            pltpu.sync_copy(zrows_v, out_hbm.at[c].at[pl.ds(off, CHUNK)])

    return deg


_prop128 = _make_prop(F)
_prop64 = _make_prop(C)
_deg = _make_deg()


def _dis_of(degb):
    return lax.rsqrt(degb[0, :, :1] + degb[1, :, :1] + 1.0)


def _tc1_body(x_ref, deg_ref, w1_ref, o_ref):
    dis = _dis_of(deg_ref[...])
    h = jnp.dot(x_ref[...], w1_ref[...], preferred_element_type=jnp.float32)
    o_ref[...] = h * dis


def _tc2_body(agg_ref, hs_ref, deg_ref, b1_ref, w2_ref, o_ref):
    dis = _dis_of(deg_ref[...])
    a = agg_ref[0] + agg_ref[1] + hs_ref[...]
    o = a * dis + b1_ref[...]
    o = jnp.maximum(o, 0.0)
    h2 = jnp.dot(o, w2_ref[...], preferred_element_type=jnp.float32)
    o_ref[...] = h2 * dis


def _tc3_body(agg_ref, hs_ref, deg_ref, b2_ref, o_ref):
    dis = _dis_of(deg_ref[...])
    a = agg_ref[0] + agg_ref[1] + hs_ref[...]
    o = a * dis + b2_ref[...]
    m = jnp.max(o, axis=1, keepdims=True)
    z = o - m
    lse = jnp.log(jnp.sum(jnp.exp(z), axis=1, keepdims=True))
    o_ref[...] = z - lse


_GRID = NPAD // BM


def _row_spec(d):
    return pl.BlockSpec((BM, d), lambda i: (i, 0))


def _p2_spec(d):
    return pl.BlockSpec((2, BM, d), lambda i: (0, i, 0))


def _full_spec(r, c_):
    return pl.BlockSpec((r, c_), lambda i: (0, 0))


_tc1 = pl.pallas_call(
    _tc1_body,
    grid=(_GRID,),
    in_specs=[_row_spec(F), _p2_spec(8), _full_spec(F, F)],
    out_specs=_row_spec(F),
    out_shape=jax.ShapeDtypeStruct((NPAD, F), jnp.float32),
)

_tc2 = pl.pallas_call(
    _tc2_body,
    grid=(_GRID,),
    in_specs=[_p2_spec(F), _row_spec(F), _p2_spec(8), _full_spec(1, F),
              _full_spec(F, C)],
    out_specs=_row_spec(C),
    out_shape=jax.ShapeDtypeStruct((NPAD, C), jnp.float32),
)

_tc3 = pl.pallas_call(
    _tc3_body,
    grid=(_GRID,),
    in_specs=[_p2_spec(C), _row_spec(C), _p2_spec(8), _full_spec(1, C)],
    out_specs=_row_spec(C),
    out_shape=jax.ShapeDtypeStruct((NPAD, C), jnp.float32),
)


@jax.jit
def kernel(x, edge_index, W1, b1, W2, b2):
    src = edge_index[0].astype(jnp.int32)
    dst = edge_index[1].astype(jnp.int32)
    ppt = EPT - NE // 32
    pad_dst = jnp.broadcast_to(
        N + jnp.arange(ppt, dtype=jnp.int32), (32, ppt))
    pad_src = jnp.zeros((32, ppt), jnp.int32)
    src3 = jnp.concatenate(
        [src.reshape(32, NE // 32), pad_src], axis=1).reshape(
            32, 2, HALF, CHUNK)
    dst3 = jnp.concatenate(
        [dst.reshape(32, NE // 32), pad_dst], axis=1).reshape(
            32, 2, HALF, CHUNK)
    xp = jnp.pad(x, ((0, NPAD - N), (0, 0)))
    z128 = jnp.zeros((CHUNK, F), jnp.float32)
    z64 = jnp.zeros((CHUNK, C), jnp.float32)
    z8 = jnp.zeros((CHUNK, 8), jnp.float32)
    ones8 = jnp.ones((CHUNK, 8), jnp.float32)

    degp = _deg(dst3, z8, ones8)
    h1s = _tc1(xp, degp, W1)
    agg1 = _prop128(h1s, src3, dst3, z128)
    h2s = _tc2(agg1, h1s, degp, b1.reshape(1, F), W2)
    agg2 = _prop64(h2s, src3, dst3, z64)
    out = _tc3(agg2, h2s, degp, b2.reshape(1, C))
    return out[:N]


# speedup vs baseline: 2.9397x; 2.6647x over previous
import functools
import jax
import jax.numpy as jnp
from jax import lax
from jax.experimental import pallas as pl
from jax.experimental.pallas import tpu as pltpu
from jax.experimental.pallas import tpu_sc as plsc

N = 10000
NPAD = 10240
F = 128
C = 64
NE = 320000
CHUNK = 128
NCHUNK = 80
HALF = NCHUNK // 2
EPT = NCHUNK * CHUNK
NE_PAD = EPT * 32
ROWS_PT = NPAD // 16
BM = 1024


def _make_prop(D):
    mesh = plsc.VectorSubcoreMesh(core_axis_name="c", subcore_axis_name="s")

    @functools.partial(
        pl.kernel,
        out_type=jax.ShapeDtypeStruct((2, NPAD, D), jnp.float32),
        mesh=mesh,
        compiler_params=pltpu.CompilerParams(use_tc_tiling_on_sc=False),
        scratch_types=[
            pltpu.VMEM((HALF, CHUNK), jnp.int32),
            pltpu.VMEM((HALF, CHUNK), jnp.int32),
            pltpu.VMEM((CHUNK, D), jnp.float32),
            pltpu.VMEM((CHUNK, D), jnp.float32),
            pltpu.VMEM_SHARED((NPAD, D), jnp.float32),
            pltpu.SemaphoreType.DMA,
            pltpu.SemaphoreType.DMA,
        ],
    )
    def prop(h_hbm, ei_hbm, z_hbm, out_hbm, src_v, dst_v, rows_a,
             rows_b, agg_sh, sem_a, sem_b):
        c = lax.axis_index("c")
        s = lax.axis_index("s")
        wid = c * 16 + s
        pltpu.sync_copy(z_hbm, rows_a)
        for j in range(ROWS_PT // CHUNK):
            pltpu.sync_copy(
                rows_a, agg_sh.at[pl.ds(s * ROWS_PT + j * CHUNK, CHUNK)])
        plsc.subcore_barrier()

        for hf in range(2):
            pltpu.sync_copy(ei_hbm.at[0].at[wid].at[hf], src_v)
            pltpu.sync_copy(ei_hbm.at[1].at[wid].at[hf], dst_v)
            pltpu.async_copy(h_hbm.at[src_v.at[0]], rows_a, sem_a)

            def body(k, carry):
                j = 2 * k
                pltpu.async_copy(h_hbm.at[src_v.at[j + 1]], rows_b, sem_b)
                pltpu.make_async_copy(
                    h_hbm.at[src_v.at[j]], rows_a, sem_a).wait()
                pltpu.sync_copy(rows_a, agg_sh.at[dst_v.at[j]], add=True)
                pltpu.async_copy(h_hbm.at[src_v.at[j + 2]], rows_a, sem_a)
                pltpu.make_async_copy(
                    h_hbm.at[src_v.at[j + 1]], rows_b, sem_b).wait()
                pltpu.sync_copy(rows_b, agg_sh.at[dst_v.at[j + 1]], add=True)
                return carry

            lax.fori_loop(0, (HALF - 2) // 2, body, 0)
            pltpu.async_copy(h_hbm.at[src_v.at[HALF - 1]], rows_b, sem_b)
            pltpu.make_async_copy(
                h_hbm.at[src_v.at[HALF - 2]], rows_a, sem_a).wait()
            pltpu.sync_copy(rows_a, agg_sh.at[dst_v.at[HALF - 2]], add=True)
            pltpu.make_async_copy(
                h_hbm.at[src_v.at[HALF - 1]], rows_b, sem_b).wait()
            pltpu.sync_copy(rows_b, agg_sh.at[dst_v.at[HALF - 1]], add=True)
        plsc.subcore_barrier()
        for j in range(ROWS_PT // CHUNK):
            off = s * ROWS_PT + j * CHUNK
            pltpu.sync_copy(agg_sh.at[pl.ds(off, CHUNK)], rows_a)
            pltpu.sync_copy(rows_a, out_hbm.at[c].at[pl.ds(off, CHUNK)])

    return prop


def _make_deg():
    D = 8
    mesh = plsc.VectorSubcoreMesh(core_axis_name="c", subcore_axis_name="s")

    @functools.partial(
        pl.kernel,
        out_type=jax.ShapeDtypeStruct((2, NPAD, D), jnp.float32),
        mesh=mesh,
        compiler_params=pltpu.CompilerParams(use_tc_tiling_on_sc=False),
        scratch_types=[
            pltpu.VMEM((HALF, CHUNK), jnp.int32),
            pltpu.VMEM((CHUNK, D), jnp.float32),
            pltpu.VMEM((CHUNK, D), jnp.float32),
            pltpu.VMEM_SHARED((NPAD, D), jnp.float32),
        ],
    )
    def deg(ei_hbm, z_hbm, ones_hbm, out_hbm, dst_v, zrows_v, ones_v, deg_sh):
        c = lax.axis_index("c")
        s = lax.axis_index("s")
        wid = c * 16 + s
        pltpu.sync_copy(z_hbm, zrows_v)
        pltpu.sync_copy(ones_hbm, ones_v)
        for j in range(ROWS_PT // CHUNK):
            pltpu.sync_copy(
                zrows_v, deg_sh.at[pl.ds(s * ROWS_PT + j * CHUNK, CHUNK)])
        plsc.subcore_barrier()

        def body(j, carry):
            pltpu.sync_copy(ones_v, deg_sh.at[dst_v.at[j]], add=True)
            return carry

        for hf in range(2):
            pltpu.sync_copy(ei_hbm.at[1].at[wid].at[hf], dst_v)
            lax.fori_loop(0, HALF, body, 0)
        plsc.subcore_barrier()
        for j in range(ROWS_PT // CHUNK):
            off = s * ROWS_PT + j * CHUNK
            pltpu.sync_copy(deg_sh.at[pl.ds(off, CHUNK)], zrows_v)
            pltpu.sync_copy(zrows_v, out_hbm.at[c].at[pl.ds(off, CHUNK)])

    return deg


_prop128 = _make_prop(F)
_prop64 = _make_prop(C)
_deg = _make_deg()


def _dis_of(degb):
    return lax.rsqrt(degb[0, :, :1] + degb[1, :, :1] + 1.0)


def _tc1_body(x_ref, deg_ref, w1_ref, o_ref):
    dis = _dis_of(deg_ref[...])
    h = jnp.dot(x_ref[...], w1_ref[...], preferred_element_type=jnp.float32)
    o_ref[...] = h * dis


def _tc2_body(agg_ref, hs_ref, deg_ref, b1_ref, w2_ref, o_ref):
    dis = _dis_of(deg_ref[...])
    a = agg_ref[0] + agg_ref[1] + hs_ref[...]
    o = a * dis + b1_ref[...]
    o = jnp.maximum(o, 0.0)
    h2 = jnp.dot(o, w2_ref[...], preferred_element_type=jnp.float32)
    o_ref[...] = h2 * dis


def _tc3_body(agg_ref, hs_ref, deg_ref, b2_ref, o_ref):
    dis = _dis_of(deg_ref[...])
    a = agg_ref[0] + agg_ref[1] + hs_ref[...]
    o = a * dis + b2_ref[...]
    m = jnp.max(o, axis=1, keepdims=True)
    z = o - m
    lse = jnp.log(jnp.sum(jnp.exp(z), axis=1, keepdims=True))
    o_ref[...] = z - lse


_GRID = NPAD // BM


def _row_spec(d):
    return pl.BlockSpec((BM, d), lambda i: (i, 0))


def _p2_spec(d):
    return pl.BlockSpec((2, BM, d), lambda i: (0, i, 0))


def _full_spec(r, c_):
    return pl.BlockSpec((r, c_), lambda i: (0, 0))


_tc1 = pl.pallas_call(
    _tc1_body,
    grid=(_GRID,),
    in_specs=[_row_spec(F), _p2_spec(8), _full_spec(F, F)],
    out_specs=_row_spec(F),
    out_shape=jax.ShapeDtypeStruct((NPAD, F), jnp.float32),
)

_tc2 = pl.pallas_call(
    _tc2_body,
    grid=(_GRID,),
    in_specs=[_p2_spec(F), _row_spec(F), _p2_spec(8), _full_spec(1, F),
              _full_spec(F, C)],
    out_specs=_row_spec(C),
    out_shape=jax.ShapeDtypeStruct((NPAD, C), jnp.float32),
)

_tc3 = pl.pallas_call(
    _tc3_body,
    grid=(_GRID,),
    in_specs=[_p2_spec(C), _row_spec(C), _p2_spec(8), _full_spec(1, C)],
    out_specs=_row_spec(C),
    out_shape=jax.ShapeDtypeStruct((NPAD, C), jnp.float32),
)


@jax.jit
def kernel(x, edge_index, W1, b1, W2, b2):
    ppt = EPT - NE // 32
    t_ids = jnp.arange(32, dtype=jnp.int32)[None, :, None]
    k_ids = jnp.arange(ppt, dtype=jnp.int32)[None, None, :]
    pad_src = (t_ids * 320 + k_ids * 13) % NPAD
    pad_dst = N + (k_ids + t_ids * 15) % ppt
    pads = jnp.concatenate([pad_src, pad_dst], axis=0)
    ei5 = jnp.concatenate(
        [edge_index.astype(jnp.int32).reshape(2, 32, NE // 32), pads],
        axis=2).reshape(2, 32, 2, HALF, CHUNK)
    xp = jnp.pad(x, ((0, NPAD - N), (0, 0)))
    z128 = jnp.zeros((CHUNK, F), jnp.float32)
    z64 = jnp.zeros((CHUNK, C), jnp.float32)
    z8 = jnp.zeros((CHUNK, 8), jnp.float32)
    ones8 = jnp.ones((CHUNK, 8), jnp.float32)

    degp = _deg(ei5, z8, ones8)
    h1s = _tc1(xp, degp, W1)
    agg1 = _prop128(h1s, ei5, z128)
    h2s = _tc2(agg1, h1s, degp, b1.reshape(1, F), W2)
    agg2 = _prop64(h2s, ei5, z64)
    out = _tc3(agg2, h2s, degp, b2.reshape(1, C))
    return out[:N]
